# baseline (speedup 1.0000x reference)
import functools

import jax
import jax.numpy as jnp
from jax import lax
from jax.experimental import pallas as pl
from jax.experimental.pallas import tpu as pltpu
from jax.experimental.pallas import tpu_sc as plsc

N_PROMPT = 10
B = 1024
L = 210
D = 128
NC, NS = 2, 16
NW = NC * NS
NLG = 4
NBC = NW // NLG
BW = B // NBC
NBLK = -(-(L + 1) // NLG)
LPAD = NLG * NBLK


def _build():
    mesh = plsc.VectorSubcoreMesh(core_axis_name="c", subcore_axis_name="s")

    @functools.partial(
        pl.kernel,
        mesh=mesh,
        out_type=jax.ShapeDtypeStruct((L, B, D), jnp.float32),
        scratch_types=[
            pltpu.VMEM((NBLK, BW), jnp.int32),
            pltpu.VMEM((BW, D), jnp.float32),
            pltpu.VMEM((BW, D), jnp.float32),
            pltpu.SemaphoreType.DMA,
            pltpu.SemaphoreType.DMA,
            pltpu.SemaphoreType.DMA,
            pltpu.SemaphoreType.DMA,
        ],
    )
    def emb(idx_hbm, wte_hbm, pt_hbm, out_hbm,
            idx_v, buf0, buf1, sem_g0, sem_g1, sem_w0, sem_w1):
        wid = lax.axis_index("s") * NC + lax.axis_index("c")
        lg = wid // NBC
        b0 = (wid % NBC) * BW
        bufs = (buf0, buf1)
        sems_g = (sem_g0, sem_g1)
        sems_w = (sem_w0, sem_w1)

        def fire_gather(i, j):
            l = lg + NLG * i

            @pl.when(l < N_PROMPT)
            def _():
                pltpu.async_copy(pt_hbm.at[idx_v.at[i]], bufs[j], sems_g[j])

            @pl.when(l >= N_PROMPT)
            def _():
                pltpu.async_copy(wte_hbm.at[idx_v.at[i]], bufs[j], sems_g[j])

        def wait_gather(i, j):
            l = lg + NLG * i

            @pl.when(l < N_PROMPT)
            def _():
                pltpu.make_async_copy(pt_hbm.at[idx_v.at[i]], bufs[j],
                                      sems_g[j]).wait()

            @pl.when(l >= N_PROMPT)
            def _():
                pltpu.make_async_copy(wte_hbm.at[idx_v.at[i]], bufs[j],
                                      sems_g[j]).wait()

        def fire_write(i, j):
            l = lg + NLG * i

            @pl.when(l < L)
            def _():
                pltpu.async_copy(bufs[j], out_hbm.at[l, pl.ds(b0, BW)],
                                 sems_w[j])

        def wait_write(j):
            pltpu.make_async_copy(bufs[j], out_hbm.at[0, pl.ds(0, BW)],
                                  sems_w[j]).wait()

        pltpu.sync_copy(idx_hbm.at[wid], idx_v)

        fire_gather(0, 0)

        def step(i, j):
            @pl.when(i >= 1)
            def _():
                wait_write(1 - j)

            @pl.when(i + 1 < NBLK)
            def _():
                fire_gather(i + 1, 1 - j)

            wait_gather(i, j)
            fire_write(i, j)

        def body(g, carry):
            step(2 * g, 0)
            step(2 * g + 1, 1)
            return carry

        lax.fori_loop(0, (NBLK - 1) // 2, body, 0)
        step(NBLK - 1, 0)

        @pl.when(lg + NLG * (NBLK - 1) < L)
        def _():
            wait_write(0)

    return emb


_EMB = _build()


def kernel(tokens, MoE_type_tensor, wte_weight,
           learned_embedding_text, learned_embedding_table):
    m10 = MoE_type_tensor.astype(jnp.int32) * N_PROMPT
    mp = m10[None, :] + jnp.arange(N_PROMPT, dtype=jnp.int32)[:, None]
    mt = tokens[:, N_PROMPT:].astype(jnp.int32).T
    mz = jnp.zeros((LPAD - L, B), jnp.int32)
    M = jnp.concatenate([mp, mt, mz], axis=0)
    A = M.reshape(NBLK, NLG, NBC, BW).transpose(1, 2, 0, 3)
    A = A.reshape(NW, NBLK, BW)
    ptable = jnp.concatenate([learned_embedding_text.astype(jnp.float32),
                              learned_embedding_table.astype(jnp.float32)],
                             axis=0)
    out = _EMB(A, wte_weight.astype(jnp.float32), ptable)
    return jnp.transpose(out, (1, 0, 2))


# speedup vs baseline: 1.2361x; 1.2361x over previous
import functools

import jax
import jax.numpy as jnp
from jax import lax
from jax.experimental import pallas as pl
from jax.experimental.pallas import tpu as pltpu
from jax.experimental.pallas import tpu_sc as plsc

N_PROMPT = 10
B = 1024
L = 210
D = 128
NC, NS = 2, 16
NW = NC * NS
NLG = 4
NBC = NW // NLG
BW = B // NBC
NBLK = -(-(L + 1) // NLG)
LPAD = NLG * NBLK
NSLOT = 6
AHEAD = 3


def _build():
    mesh = plsc.VectorSubcoreMesh(core_axis_name="c", subcore_axis_name="s")

    @functools.partial(
        pl.kernel,
        mesh=mesh,
        out_type=jax.ShapeDtypeStruct((L, B, D), jnp.float32),
        scratch_types=[
            pltpu.VMEM((NBLK, BW), jnp.int32),
            [pltpu.VMEM((BW, D), jnp.float32)] * NSLOT,
            [pltpu.SemaphoreType.DMA] * NSLOT,
            [pltpu.SemaphoreType.DMA] * NSLOT,
        ],
    )
    def emb(idx_hbm, wte_hbm, pt_hbm, out_hbm,
            idx_v, bufs, sems_g, sems_w):
        wid = lax.axis_index("s") * NC + lax.axis_index("c")
        lg = wid // NBC
        b0 = (wid % NBC) * BW

        def fire_gather(i, j):
            l = lg + NLG * i

            @pl.when(l < N_PROMPT)
            def _():
                pltpu.async_copy(pt_hbm.at[idx_v.at[i]], bufs[j], sems_g[j])

            @pl.when(l >= N_PROMPT)
            def _():
                pltpu.async_copy(wte_hbm.at[idx_v.at[i]], bufs[j], sems_g[j])

        def wait_gather(i, j):
            l = lg + NLG * i

            @pl.when(l < N_PROMPT)
            def _():
                pltpu.make_async_copy(pt_hbm.at[idx_v.at[i]], bufs[j],
                                      sems_g[j]).wait()

            @pl.when(l >= N_PROMPT)
            def _():
                pltpu.make_async_copy(wte_hbm.at[idx_v.at[i]], bufs[j],
                                      sems_g[j]).wait()

        def fire_write(i, j):
            l = lg + NLG * i

            @pl.when(l < L)
            def _():
                pltpu.async_copy(bufs[j], out_hbm.at[l, pl.ds(b0, BW)],
                                 sems_w[j])

        def wait_write(j):
            pltpu.make_async_copy(bufs[j], out_hbm.at[0, pl.ds(0, BW)],
                                  sems_w[j]).wait()

        pltpu.sync_copy(idx_hbm.at[wid], idx_v)

        for f in range(AHEAD):
            fire_gather(f, f % NSLOT)

        def step(i, j):
            f = i + AHEAD
            jf = (j + AHEAD) % NSLOT

            @pl.when(f < NBLK)
            def _():
                @pl.when(f >= NSLOT)
                def _():
                    wait_write(jf)

                fire_gather(f, jf)

            wait_gather(i, j)
            fire_write(i, j)

        def body(g, carry):
            for j in range(NSLOT):
                step(NSLOT * g + j, j)
            return carry

        lax.fori_loop(0, NBLK // NSLOT, body, 0)
        for i in range(NSLOT * (NBLK // NSLOT), NBLK):
            step(i, i % NSLOT)

        for i in range(NBLK - NSLOT, NBLK):
            @pl.when(lg + NLG * i < L)
            def _():
                wait_write(i % NSLOT)

    return emb


_EMB = _build()


def kernel(tokens, MoE_type_tensor, wte_weight,
           learned_embedding_text, learned_embedding_table):
    m10 = MoE_type_tensor.astype(jnp.int32) * N_PROMPT
    mp = m10[None, :] + jnp.arange(N_PROMPT, dtype=jnp.int32)[:, None]
    mt = tokens[:, N_PROMPT:].astype(jnp.int32).T
    mz = jnp.zeros((LPAD - L, B), jnp.int32)
    M = jnp.concatenate([mp, mt, mz], axis=0)
    A = M.reshape(NBLK, NLG, NBC, BW).transpose(1, 2, 0, 3)
    A = A.reshape(NW, NBLK, BW)
    ptable = jnp.concatenate([learned_embedding_text.astype(jnp.float32),
                              learned_embedding_table.astype(jnp.float32)],
                             axis=0)
    out = _EMB(A, wte_weight.astype(jnp.float32), ptable)
    return jnp.transpose(out, (1, 0, 2))


# speedup vs baseline: 1.3820x; 1.1181x over previous
import functools

import jax
import jax.numpy as jnp
from jax import lax
from jax.experimental import pallas as pl
from jax.experimental.pallas import tpu as pltpu
from jax.experimental.pallas import tpu_sc as plsc

N_PROMPT = 10
B = 1024
L = 210
D = 128
NC, NS = 2, 16
NW = NC * NS
NLG = 4
NBC = NW // NLG
BW = B // NBC
NBLK = -(-(L + 1) // NLG)
LPAD = NLG * NBLK
NSLOT = 6
AHEAD = 3
PREP = 64


def _build():
    mesh = plsc.VectorSubcoreMesh(core_axis_name="c", subcore_axis_name="s")

    @functools.partial(
        pl.kernel,
        mesh=mesh,
        out_type=jax.ShapeDtypeStruct((L, B, D), jnp.float32),
        scratch_types=[
            pltpu.VMEM((NBLK, BW), jnp.int32),
            [pltpu.VMEM((BW, D), jnp.float32)] * NSLOT,
            [pltpu.SemaphoreType.DMA] * NSLOT,
            [pltpu.SemaphoreType.DMA] * NSLOT,
        ],
    )
    def emb(idx_hbm, wte_hbm, pt_hbm, out_hbm,
            idx_v, bufs, sems_g, sems_w):
        wid = lax.axis_index("s") * NC + lax.axis_index("c")
        lg = wid // NBC
        b0 = (wid % NBC) * BW

        def fire_gather(i, j):
            l = lg + NLG * i

            @pl.when(l < N_PROMPT)
            def _():
                pltpu.async_copy(pt_hbm.at[idx_v.at[i]], bufs[j], sems_g[j])

            @pl.when(l >= N_PROMPT)
            def _():
                pltpu.async_copy(wte_hbm.at[idx_v.at[i]], bufs[j], sems_g[j])

        def wait_gather(i, j):
            l = lg + NLG * i

            @pl.when(l < N_PROMPT)
            def _():
                pltpu.make_async_copy(pt_hbm.at[idx_v.at[i]], bufs[j],
                                      sems_g[j]).wait()

            @pl.when(l >= N_PROMPT)
            def _():
                pltpu.make_async_copy(wte_hbm.at[idx_v.at[i]], bufs[j],
                                      sems_g[j]).wait()

        def fire_write(i, j):
            l = lg + NLG * i

            @pl.when(l < L)
            def _():
                pltpu.async_copy(bufs[j], out_hbm.at[l, pl.ds(b0, BW)],
                                 sems_w[j])

        def wait_write(j):
            pltpu.make_async_copy(bufs[j], out_hbm.at[0, pl.ds(0, BW)],
                                  sems_w[j]).wait()

        pltpu.sync_copy(idx_hbm.at[wid], idx_v)

        for f in range(AHEAD):
            fire_gather(f, f % NSLOT)

        def step(i, j):
            f = i + AHEAD
            jf = (j + AHEAD) % NSLOT

            @pl.when(f < NBLK)
            def _():
                @pl.when(f >= NSLOT)
                def _():
                    wait_write(jf)

                fire_gather(f, jf)

            wait_gather(i, j)
            fire_write(i, j)

        def body(g, carry):
            for j in range(NSLOT):
                step(NSLOT * g + j, j)
            return carry

        lax.fori_loop(0, NBLK // NSLOT, body, 0)
        for i in range(NSLOT * (NBLK // NSLOT), NBLK):
            step(i, i % NSLOT)

        for i in range(NBLK - NSLOT, NBLK):
            @pl.when(lg + NLG * i < L)
            def _():
                wait_write(i % NSLOT)

    return emb


_EMB = _build()


def kernel(tokens, MoE_type_tensor, wte_weight,
           learned_embedding_text, learned_embedding_table):
    m10 = MoE_type_tensor.astype(jnp.int32) * N_PROMPT
    rep = 2 * N_PROMPT * (jnp.arange(B, dtype=jnp.int32) % PREP)
    mp = (m10 + rep)[None, :] + jnp.arange(N_PROMPT, dtype=jnp.int32)[:, None]
    mt = tokens[:, N_PROMPT:].astype(jnp.int32).T
    mz = jnp.zeros((LPAD - L, B), jnp.int32)
    M = jnp.concatenate([mp, mt, mz], axis=0)
    A = M.reshape(NBLK, NLG, NBC, BW).transpose(1, 2, 0, 3)
    A = A.reshape(NW, NBLK, BW)
    ptable = jnp.concatenate([learned_embedding_text.astype(jnp.float32),
                              learned_embedding_table.astype(jnp.float32)],
                             axis=0)
    ptable = jnp.tile(ptable, (PREP, 1))
    out = _EMB(A, wte_weight.astype(jnp.float32), ptable)
    return jnp.transpose(out, (1, 0, 2))
